# baseline (speedup 1.0000x reference)
import jax
import jax.numpy as jnp
from jax.experimental import pallas as pl

B, T, N, F = 4, 16, 300, 158
H, HEADS, DH = 256, 4, 64
NEG = -1e30


def _elu(x):
    return jnp.where(x > 0, x, jnp.exp(jnp.minimum(x, 0.0)) - 1.0)


def _acat(as_ref, ad_ref):
    r = jax.lax.broadcasted_iota(jnp.int32, (2 * HEADS, H), 0)
    c = jax.lax.broadcasted_iota(jnp.int32, (2 * HEADS, H), 1)
    head2 = 2 * (c // DH)
    acat_t = (jnp.where(r == head2, as_ref[...], 0.0)
              + jnp.where(r == head2 + 1, ad_ref[...], 0.0))
    return acat_t.T


def _gat_layer(h, madd, W_ref, acat, ones_col):
    xp = jnp.dot(h, W_ref[...].T, preferred_element_type=jnp.float32)
    al = jnp.dot(xp, acat, preferred_element_type=jnp.float32)
    al_b = (al * 1.4426950408889634).astype(jnp.bfloat16)
    al_t = al_b.T
    outs = []
    for hd in range(HEADS):
        xph = xp[:, hd * DH:(hd + 1) * DH]
        xph_aug = jnp.concatenate(
            [xph.astype(jnp.bfloat16), ones_col], axis=1)
        u = al_b[:, 2 * hd + 1:2 * hd + 2] + al_t[2 * hd:2 * hd + 1, :]
        l = jnp.maximum(u, jnp.bfloat16(0.2) * u)
        ex = jnp.exp2(l + madd)
        agg = jnp.dot(ex, xph_aug, preferred_element_type=jnp.float32)
        outs.append(agg[:, :DH] * (1.0 / (agg[:, DH:DH + 1] + 1e-16)))
    return outs


def _fused_kernel(x_ref, adj_ref, Win_ref, bin_ref, lng_ref, lnb_ref,
                  W0_ref, as0_ref, ad0_ref, b0_ref,
                  W1_ref, as1_ref, ad1_ref, b1_ref,
                  Wout_ref, bout_ref, out_ref):
  acat0 = _acat(as0_ref, ad0_ref)
  acat1 = _acat(as1_ref, ad1_ref)
  for g in range(2):
    x = x_ref[g]
    h = jnp.dot(x, Win_ref[...].T, preferred_element_type=jnp.float32) + bin_ref[...]
    mu = jnp.mean(h, axis=1, keepdims=True)
    d = h - mu
    var = jnp.mean(d * d, axis=1, keepdims=True)
    h = d * jax.lax.rsqrt(var + 1e-5) * lng_ref[...] + lnb_ref[...]
    h = _elu(h)

    adj = adj_ref[g].astype(jnp.int32)
    row = jax.lax.broadcasted_iota(jnp.int32, (N, N), 0)
    col = jax.lax.broadcasted_iota(jnp.int32, (N, N), 1)
    madd = jnp.where((adj != 0) | (row == col), 0.0, NEG).T
    madd = madd.astype(jnp.bfloat16)
    ones_col = jnp.ones((N, 1), jnp.bfloat16)

    o0 = _gat_layer(h, madd, W0_ref, acat0, ones_col)
    o0 = jnp.concatenate(o0, axis=1) + b0_ref[...]
    h = h + _elu(o0)

    o1 = _gat_layer(h, madd, W1_ref, acat1, ones_col)
    o1 = (o1[0] + o1[1] + o1[2] + o1[3]) * 0.25 + b1_ref[...]

    out_ref[g] = jnp.dot(o1, Wout_ref[...].T, preferred_element_type=jnp.float32) + bout_ref[...]


@jax.jit
def kernel(x_alpha, sector_graph, W_in, b_in, ln_g, ln_b, W0, att_src0,
           att_dst0, bias0, W1, att_src1, att_dst1, bias1, W_out, b_out):
    x_last = x_alpha[:, -1]
    adj8 = sector_graph.astype(jnp.int8)
    full = lambda *shape: pl.BlockSpec(shape, lambda b: (0,) * len(shape))
    grid_spec = pl.GridSpec(
        grid=(2,),
        in_specs=[
            pl.BlockSpec((2, N, F), lambda b: (b, 0, 0)),
            pl.BlockSpec((2, N, N), lambda b: (b, 0, 0)),
            full(H, F), full(1, H), full(1, H), full(1, H),
            full(HEADS * DH, H), full(1, H), full(1, H), full(1, HEADS * DH),
            full(HEADS * DH, H), full(1, H), full(1, H), full(1, DH),
            full(H, DH), full(1, H),
        ],
        out_specs=pl.BlockSpec((2, N, H), lambda b: (b, 0, 0)),
    )
    return pl.pallas_call(
        _fused_kernel,
        grid_spec=grid_spec,
        out_shape=jax.ShapeDtypeStruct((B, N, H), jnp.float32),
    )(x_last, adj8, W_in, b_in.reshape(1, H), ln_g.reshape(1, H),
      ln_b.reshape(1, H), W0, att_src0.reshape(1, H), att_dst0.reshape(1, H),
      bias0.reshape(1, HEADS * DH), W1, att_src1.reshape(1, H),
      att_dst1.reshape(1, H), bias1.reshape(1, DH), W_out, b_out.reshape(1, H))


# speedup vs baseline: 1.0081x; 1.0081x over previous
import jax
import jax.numpy as jnp
from jax.experimental import pallas as pl

B, T, N, F = 4, 16, 300, 158
H, HEADS, DH = 256, 4, 64
NEG = -1e30


def _elu(x):
    return jnp.where(x > 0, x, jnp.exp(jnp.minimum(x, 0.0)) - 1.0)


def _acat(a_s_row, a_d_row):
    r = jax.lax.broadcasted_iota(jnp.int32, (2 * HEADS, H), 0)
    c = jax.lax.broadcasted_iota(jnp.int32, (2 * HEADS, H), 1)
    head2 = 2 * (c // DH)
    acat_t = (jnp.where(r == head2, a_s_row, 0.0)
              + jnp.where(r == head2 + 1, a_d_row, 0.0))
    return acat_t.T


def _gat_layer(h, madd, W, acat, ones_col):
    xp = jnp.dot(h, W.T, preferred_element_type=jnp.float32)
    al = jnp.dot(xp, acat, preferred_element_type=jnp.float32)
    al_b = (al * 1.4426950408889634).astype(jnp.bfloat16)
    al_t = al_b.T
    outs = []
    for hd in range(HEADS):
        xph = xp[:, hd * DH:(hd + 1) * DH]
        xph_aug = jnp.concatenate(
            [xph.astype(jnp.bfloat16), ones_col], axis=1)
        u = al_b[:, 2 * hd + 1:2 * hd + 2] + al_t[2 * hd:2 * hd + 1, :]
        l = jnp.maximum(u, jnp.bfloat16(0.2) * u)
        ex = jnp.exp2(l + madd)
        agg = jnp.dot(ex, xph_aug, preferred_element_type=jnp.float32)
        outs.append(agg[:, :DH] * (1.0 / (agg[:, DH:DH + 1] + 1e-16)))
    return outs


def _fused_kernel(x_ref, adj_ref, wm_ref, wv_ref, out_ref):
  wm = wm_ref[...]
  wv = wv_ref[...]
  acat0 = _acat(wv[6:7], wv[7:8])
  acat1 = _acat(wv[8:9], wv[9:10])
  for g in range(2):
    x = x_ref[g]
    h = jnp.dot(x, wm[:, 512:512 + F].T, preferred_element_type=jnp.float32) + wv[0:1]
    mu = jnp.mean(h, axis=1, keepdims=True)
    d = h - mu
    var = jnp.mean(d * d, axis=1, keepdims=True)
    h = d * jax.lax.rsqrt(var + 1e-5) * wv[1:2] + wv[2:3]
    h = _elu(h)

    adj = adj_ref[g].astype(jnp.int32)
    row = jax.lax.broadcasted_iota(jnp.int32, (N, N), 0)
    col = jax.lax.broadcasted_iota(jnp.int32, (N, N), 1)
    madd = jnp.where((adj != 0) | (row == col), 0.0, NEG).T
    madd = madd.astype(jnp.bfloat16)
    ones_col = jnp.ones((N, 1), jnp.bfloat16)

    o0 = _gat_layer(h, madd, wm[:, 0:256], acat0, ones_col)
    o0 = jnp.concatenate(o0, axis=1) + wv[3:4]
    h = h + _elu(o0)

    o1 = _gat_layer(h, madd, wm[:, 256:512], acat1, ones_col)
    o1 = (o1[0] + o1[1] + o1[2] + o1[3]) * 0.25 + wv[5:6, :DH]

    out_ref[g] = jnp.dot(o1, wm[:, 768:832].T, preferred_element_type=jnp.float32) + wv[4:5]


@jax.jit
def kernel(x_alpha, sector_graph, W_in, b_in, ln_g, ln_b, W0, att_src0,
           att_dst0, bias0, W1, att_src1, att_dst1, bias1, W_out, b_out):
    x_last = x_alpha[:, -1]
    adj8 = sector_graph.astype(jnp.int8)
    full = lambda *shape: pl.BlockSpec(shape, lambda b: (0,) * len(shape))
    grid_spec = pl.GridSpec(
        grid=(2,),
        in_specs=[
            pl.BlockSpec((2, N, F), lambda b: (b, 0, 0)),
            pl.BlockSpec((2, N, N), lambda b: (b, 0, 0)),
            full(H, 832), full(10, H),
        ],
        out_specs=pl.BlockSpec((2, N, H), lambda b: (b, 0, 0)),
    )
    wm = jnp.concatenate(
        [W0, W1, W_in, jnp.zeros((H, 256 - F), jnp.float32), W_out], axis=1)
    wv = jnp.concatenate(
        [b_in.reshape(1, H), ln_g.reshape(1, H), ln_b.reshape(1, H),
         bias0.reshape(1, H), b_out.reshape(1, H),
         jnp.concatenate([bias1.reshape(1, DH),
                          jnp.zeros((1, H - DH), jnp.float32)], axis=1),
         att_src0.reshape(1, H), att_dst0.reshape(1, H),
         att_src1.reshape(1, H), att_dst1.reshape(1, H)], axis=0)
    return pl.pallas_call(
        _fused_kernel,
        grid_spec=grid_spec,
        out_shape=jax.ShapeDtypeStruct((B, N, H), jnp.float32),
    )(x_last, adj8, wm, wv)


# speedup vs baseline: 1.0095x; 1.0014x over previous
import jax
import jax.numpy as jnp
from jax.experimental import pallas as pl

B, T, N, F = 4, 16, 300, 158
H, HEADS, DH = 256, 4, 64
NEG = -1e30


def _elu(x):
    return jnp.where(x > 0, x, jnp.exp(jnp.minimum(x, 0.0)) - 1.0)


def _acat(a_s_row, a_d_row):
    r = jax.lax.broadcasted_iota(jnp.int32, (2 * HEADS, H), 0)
    c = jax.lax.broadcasted_iota(jnp.int32, (2 * HEADS, H), 1)
    head2 = 2 * (c // DH)
    acat_t = (jnp.where(r == head2, a_s_row, 0.0)
              + jnp.where(r == head2 + 1, a_d_row, 0.0))
    return acat_t.T


def _gat_layer(h, madd, W, acat, ones_col):
    xp = jnp.dot(h, W.T, preferred_element_type=jnp.float32)
    al = jnp.dot(xp, acat, preferred_element_type=jnp.float32)
    al_b = (al * 1.4426950408889634).astype(jnp.bfloat16)
    al_t = al_b.T
    outs = []
    for hd in range(HEADS):
        xph = xp[:, hd * DH:(hd + 1) * DH]
        xph_aug = jnp.concatenate(
            [xph.astype(jnp.bfloat16), ones_col], axis=1)
        u = al_b[:, 2 * hd + 1:2 * hd + 2] + al_t[2 * hd:2 * hd + 1, :]
        l = jnp.maximum(u, jnp.bfloat16(0.2) * u)
        ex = jnp.exp2(l + madd)
        agg = jnp.dot(ex, xph_aug, preferred_element_type=jnp.float32)
        outs.append(agg[:, :DH] * (1.0 / (agg[:, DH:DH + 1] + 1e-16)))
    return outs


def _fused_kernel(x_ref, adj_ref, wm_ref, wv_ref, out_ref):
  wm = wm_ref[...]
  wv = wv_ref[...]
  acat0 = _acat(wv[6:7], wv[7:8])
  acat1 = _acat(wv[8:9], wv[9:10])
  for g in range(2):
    x = x_ref[g].astype(jnp.float32)
    h = jnp.dot(x, wm[:, 512:512 + F].T, preferred_element_type=jnp.float32) + wv[0:1]
    mu = jnp.mean(h, axis=1, keepdims=True)
    d = h - mu
    var = jnp.mean(d * d, axis=1, keepdims=True)
    h = d * jax.lax.rsqrt(var + 1e-5) * wv[1:2] + wv[2:3]
    h = _elu(h)

    adj = adj_ref[g].astype(jnp.int32)
    madd = jnp.where(adj != 0, 0.0, NEG).T
    madd = madd.astype(jnp.bfloat16)
    ones_col = jnp.ones((N, 1), jnp.bfloat16)

    o0 = _gat_layer(h, madd, wm[:, 0:256], acat0, ones_col)
    o0 = jnp.concatenate(o0, axis=1) + wv[3:4]
    h = h + _elu(o0)

    o1 = _gat_layer(h, madd, wm[:, 256:512], acat1, ones_col)
    o1 = (o1[0] + o1[1] + o1[2] + o1[3]) * 0.25 + wv[5:6, :DH]

    out_ref[g] = jnp.dot(o1, wm[:, 768:832].T, preferred_element_type=jnp.float32) + wv[4:5]


@jax.jit
def kernel(x_alpha, sector_graph, W_in, b_in, ln_g, ln_b, W0, att_src0,
           att_dst0, bias0, W1, att_src1, att_dst1, bias1, W_out, b_out):
    x_last = x_alpha[:, -1].astype(jnp.bfloat16)
    eye = jnp.eye(N, dtype=jnp.bool_)
    adj8 = ((sector_graph != 0) | eye).astype(jnp.int8)
    full = lambda *shape: pl.BlockSpec(shape, lambda b: (0,) * len(shape))
    grid_spec = pl.GridSpec(
        grid=(2,),
        in_specs=[
            pl.BlockSpec((2, N, F), lambda b: (b, 0, 0)),
            pl.BlockSpec((2, N, N), lambda b: (b, 0, 0)),
            full(H, 832), full(10, H),
        ],
        out_specs=pl.BlockSpec((2, N, H), lambda b: (b, 0, 0)),
    )
    wm = jnp.concatenate(
        [W0, W1, W_in, jnp.zeros((H, 256 - F), jnp.float32), W_out], axis=1)
    wv = jnp.concatenate(
        [b_in.reshape(1, H), ln_g.reshape(1, H), ln_b.reshape(1, H),
         bias0.reshape(1, H), b_out.reshape(1, H),
         jnp.concatenate([bias1.reshape(1, DH),
                          jnp.zeros((1, H - DH), jnp.float32)], axis=1),
         att_src0.reshape(1, H), att_dst0.reshape(1, H),
         att_src1.reshape(1, H), att_dst1.reshape(1, H)], axis=0)
    return pl.pallas_call(
        _fused_kernel,
        grid_spec=grid_spec,
        out_shape=jax.ShapeDtypeStruct((B, N, H), jnp.float32),
    )(x_last, adj8, wm, wv)
